# baseline (speedup 1.0000x reference)
import jax
import jax.numpy as jnp
from jax import lax
from jax.experimental import pallas as pl
from jax.experimental.pallas import tpu as pltpu
from jax.experimental.pallas import tpu_sc as plsc

N = 10000
E = 320000
D = 128
H = 128
B = 64
OUT = 64

NC = 2
NS = 16
NW = NC * NS
K = 64
K2 = 128
NCHUNK = 160
EPW = NCHUNK * K
EPAD = NW * EPW
LAST_TRIPS = (E - (NW - 1) * EPW) // K
NACC2 = N + 16
ROWS_PT = 624
TAIL_OFF = ROWS_PT * NS
TAIL = N - TAIL_OFF

_HIGH = lax.Precision.HIGHEST


def _make_seg_body(ph, k, with_cnt):
    nchunk = EPW // k
    last_trips = (E - (NW - 1) * EPW) // k
    def body(*refs):
        if with_cnt:
            (table, src2, dst2, z128, z1, acc_out, cnt_out,
             srcv, dstv, r0, r1, cntp, sem0, sem1, acc) = refs
        else:
            (table, src2, dst2, z128, acc_out,
             srcv, dstv, r0, r1, sem0, sem1, acc) = refs
        c = lax.axis_index("c")
        s = lax.axis_index("s")
        wid = s * NC + c
        roff = s * ROWS_PT
        pltpu.sync_copy(z128.at[pl.ds(roff, ROWS_PT)],
                        acc.at[pl.ds(roff, ROWS_PT)])
        if with_cnt:
            pltpu.sync_copy(z1, cntp)
            ones = jnp.ones((16,), jnp.float32)

        @pl.when(s == NS - 1)
        def _():
            pltpu.sync_copy(z128.at[pl.ds(TAIL_OFF, TAIL)],
                            acc.at[pl.ds(TAIL_OFF, TAIL)])

        plsc.subcore_barrier()

        trips = jnp.where(wid == NW - 1, last_trips, nchunk)

        for p in range(nchunk // ph):
            pltpu.sync_copy(src2.at[wid, pl.ds(p * ph, ph)], srcv)
            pltpu.sync_copy(dst2.at[wid, pl.ds(p * ph, ph)], dstv)
            t_p = jnp.clip(trips - p * ph, 0, ph)

            @pl.when(t_p > 0)
            def _():
                pltpu.async_copy(table.at[srcv.at[0]], r0, sem0)

            def pair(i, carry):
                j0 = 2 * i
                cp1 = pltpu.async_copy(table.at[srcv.at[j0 + 1]], r1, sem1)
                pltpu.make_async_copy(table.at[srcv.at[j0]], r0, sem0).wait()
                pltpu.sync_copy(r0, acc.at[dstv.at[j0]], add=True)

                @pl.when(j0 + 2 < t_p)
                def _():
                    pltpu.async_copy(table.at[srcv.at[j0 + 2]], r0, sem0)

                if with_cnt:
                    for q in range(k // 16):
                        idx = dstv[j0, pl.ds(q * 16, 16)]
                        plsc.addupdate_scatter(cntp, [idx], ones)
                cp1.wait()
                pltpu.sync_copy(r1, acc.at[dstv.at[j0 + 1]], add=True)
                if with_cnt:
                    for q in range(k // 16):
                        idx = dstv[j0 + 1, pl.ds(q * 16, 16)]
                        plsc.addupdate_scatter(cntp, [idx], ones)
                return carry

            lax.fori_loop(0, t_p // 2, pair, 0)
        plsc.subcore_barrier()
        pltpu.sync_copy(acc.at[pl.ds(roff, ROWS_PT)],
                        acc_out.at[c, pl.ds(roff, ROWS_PT)])
        if with_cnt:
            pltpu.sync_copy(cntp, cnt_out.at[wid, 0])

        @pl.when(s == NS - 1)
        def _():
            pltpu.sync_copy(acc.at[pl.ds(TAIL_OFF, TAIL)],
                            acc_out.at[c, pl.ds(TAIL_OFF, TAIL)])

    return body


def _make_seg4_body(ph, k):
    nchunk = EPW // k
    last_trips = (E - (NW - 1) * EPW) // k

    def body(table, src2, dst2, z128, acc_out, srcv, dstv,
             r0, r1, r2, r3, s0, s1, s2, s3, acc):
        bufs = [r0, r1, r2, r3]
        sems = [s0, s1, s2, s3]
        c = lax.axis_index("c")
        s = lax.axis_index("s")
        wid = s * NC + c
        roff = s * ROWS_PT
        pltpu.sync_copy(z128.at[pl.ds(roff, ROWS_PT)],
                        acc.at[pl.ds(roff, ROWS_PT)])

        @pl.when(s == NS - 1)
        def _():
            pltpu.sync_copy(z128.at[pl.ds(TAIL_OFF, TAIL)],
                            acc.at[pl.ds(TAIL_OFF, TAIL)])

        plsc.subcore_barrier()
        trips = jnp.where(wid == NW - 1, last_trips, nchunk)

        for p in range(nchunk // ph):
            pltpu.sync_copy(src2.at[wid, pl.ds(p * ph, ph)], srcv)
            pltpu.sync_copy(dst2.at[wid, pl.ds(p * ph, ph)], dstv)
            t_p = jnp.clip(trips - p * ph, 0, ph)

            for j in range(4):
                @pl.when(j < t_p)
                def _(j=j):
                    pltpu.async_copy(table.at[srcv.at[j]], bufs[j], sems[j])

            def group(i, carry):
                for q in range(4):
                    j = 4 * i + q
                    pltpu.make_async_copy(table.at[srcv.at[j]],
                                          bufs[q], sems[q]).wait()
                    pltpu.sync_copy(bufs[q], acc.at[dstv.at[j]], add=True)

                    @pl.when(j + 4 < t_p)
                    def _(q=q, j=j):
                        pltpu.async_copy(table.at[srcv.at[j + 4]],
                                         bufs[q], sems[q])
                return carry

            lax.fori_loop(0, t_p // 4, group, 0)
        plsc.subcore_barrier()
        pltpu.sync_copy(acc.at[pl.ds(roff, ROWS_PT)],
                        acc_out.at[c, pl.ds(roff, ROWS_PT)])

        @pl.when(s == NS - 1)
        def _():
            pltpu.sync_copy(acc.at[pl.ds(TAIL_OFF, TAIL)],
                            acc_out.at[c, pl.ds(TAIL_OFF, TAIL)])

    return body


def _make_sc_kernels():
    mesh = plsc.VectorSubcoreMesh(
        core_axis_name="c", subcore_axis_name="s",
        num_cores=NC, num_subcores=NS,
    )
    ph1 = 40
    seg1 = pl.kernel(
        _make_seg_body(ph1, K, True),
        out_type=(jax.ShapeDtypeStruct((NC, N, D), jnp.float32),
                  jax.ShapeDtypeStruct((NW, 1, NACC2), jnp.float32)),
        mesh=mesh,
        compiler_params=pltpu.CompilerParams(needs_layout_passes=False),
        scratch_types=[
            pltpu.VMEM((ph1, K), jnp.int32),
            pltpu.VMEM((ph1, K), jnp.int32),
            pltpu.VMEM((K, D), jnp.float32),
            pltpu.VMEM((K, D), jnp.float32),
            pltpu.VMEM((NACC2,), jnp.float32),
            pltpu.SemaphoreType.DMA,
            pltpu.SemaphoreType.DMA,
            pltpu.VMEM_SHARED((N, D), jnp.float32),
        ],
        name="seg_cnt",
    )
    ph2 = 8
    seg2 = pl.kernel(
        _make_seg4_body(ph2, K),
        out_type=jax.ShapeDtypeStruct((NC, N, D), jnp.float32),
        mesh=mesh,
        scratch_types=[
            pltpu.VMEM((ph2, K), jnp.int32),
            pltpu.VMEM((ph2, K), jnp.int32),
            pltpu.VMEM((K, D), jnp.float32),
            pltpu.VMEM((K, D), jnp.float32),
            pltpu.VMEM((K, D), jnp.float32),
            pltpu.VMEM((K, D), jnp.float32),
            pltpu.SemaphoreType.DMA,
            pltpu.SemaphoreType.DMA,
            pltpu.SemaphoreType.DMA,
            pltpu.SemaphoreType.DMA,
            pltpu.VMEM_SHARED((N, D), jnp.float32),
        ],
        name="seg_sum",
    )
    return seg1, seg2


_sc_cache = []


def _sc_kernels():
    if not _sc_cache:
        _sc_cache.append(_make_sc_kernels())
    return _sc_cache[0]


_GRID = 10
_BN = N // _GRID


def _xr_body(x_ref, w_ref, b_ref, xr_ref):
    xr_ref[...] = lax.dot_general(x_ref[...], w_ref[...], (((1,), (1,)), ((), ())),
                                  precision=_HIGH) + b_ref[...]


def _mid_body(a_ref, c_ref, xr1_ref, w1l_ref, h1_ref):
    cnt = jnp.sum(c_ref[...], axis=1)
    inv = 1.0 / jnp.maximum(cnt, 1.0)
    mean = (a_ref[0] + a_ref[1]) * inv[:, None]
    h1 = lax.dot_general(mean, w1l_ref[...], (((1,), (1,)), ((), ())),
                         precision=_HIGH) + xr1_ref[...]
    h1_ref[...] = jnp.maximum(h1, 0.0)


def _tail_body(a_ref, c_ref, xr2_ref, w2l_ref, batch_ref, embed_ref,
               wa_ref, wb_ref, blin_ref, out_ref, pool_ref, cb_ref):
    i = pl.program_id(0)
    cnt = jnp.sum(c_ref[...], axis=1)
    inv = 1.0 / jnp.maximum(cnt, 1.0)
    mean = (a_ref[0] + a_ref[1]) * inv[:, None]
    h2 = lax.dot_general(mean, w2l_ref[...], (((1,), (1,)), ((), ())),
                         precision=_HIGH) + xr2_ref[...]
    h2 = jnp.maximum(h2, 0.0)
    ids = lax.broadcasted_iota(jnp.int32, (B, _BN), 0)
    oneh = (ids == batch_ref[0]).astype(jnp.float32)
    pool_p = lax.dot_general(oneh, h2, (((1,), (0,)), ((), ())),
                             precision=_HIGH)
    cb_p = jnp.sum(oneh, axis=1, keepdims=True)

    @pl.when(i == 0)
    def _():
        pool_ref[...] = jnp.zeros_like(pool_ref)
        cb_ref[...] = jnp.zeros_like(cb_ref)

    pool_ref[...] += pool_p
    cb_ref[...] += cb_p

    @pl.when(i == _GRID - 1)
    def _():
        pooled = pool_ref[...] / jnp.maximum(cb_ref[...], 1.0)
        out = lax.dot_general(pooled, wa_ref[...], (((1,), (1,)), ((), ())),
                              precision=_HIGH)
        out = out + lax.dot_general(embed_ref[...], wb_ref[...],
                                    (((1,), (1,)), ((), ())), precision=_HIGH)
        out_ref[...] = out + blin_ref[...]


def kernel(x, edge_index, batch, embed, W1l, b1l, W1r, W2l, b2l, W2r, Wlin, blin):
    npad = EPAD - E
    srcf = jnp.concatenate([edge_index[0], jnp.zeros((npad,), jnp.int32)])
    dstf = jnp.concatenate([edge_index[1], jnp.zeros((npad,), jnp.int32)])
    src2 = srcf.reshape(NW, NCHUNK, K)
    dst2 = dstf.reshape(NW, NCHUNK, K)
    z128 = jnp.zeros((N, D), jnp.float32)
    z1 = jnp.zeros((NACC2,), jnp.float32)
    b1 = b1l.reshape(1, H)
    b2 = b2l.reshape(1, H)
    bl = blin.reshape(1, OUT)
    batch3 = batch.reshape(_GRID, 1, _BN)
    wa = Wlin[:, :H]
    wb = Wlin[:, H:]

    seg1, seg2 = _sc_kernels()

    blk_a = pl.BlockSpec((NC, _BN, D), lambda i: (0, i, 0))
    blk_c = pl.BlockSpec((_BN, NW), lambda i: (i, 0))
    blk_r = pl.BlockSpec((_BN, D), lambda i: (i, 0))
    blk_w = pl.BlockSpec((H, H), lambda i: (0, 0))
    blk_b = pl.BlockSpec((1, H), lambda i: (0, 0))

    def xr_call(inp, w, bias):
        return pl.pallas_call(
            _xr_body,
            grid=(_GRID,),
            in_specs=[blk_r, blk_w, blk_b],
            out_specs=blk_r,
            out_shape=jax.ShapeDtypeStruct((N, D), jnp.float32),
        )(inp, w, bias)

    a1, cntw3 = seg1(x, src2, dst2, z128, z1)
    xr1 = xr_call(x, W1r, b1)
    cntw = jnp.transpose(cntw3.reshape(NW, NACC2))

    h1 = pl.pallas_call(
        _mid_body,
        grid=(_GRID,),
        in_specs=[blk_a, blk_c, blk_r, blk_w],
        out_specs=blk_r,
        out_shape=jax.ShapeDtypeStruct((N, D), jnp.float32),
    )(a1, cntw, xr1, W1l)

    a2 = seg2(h1, src2, dst2, z128)
    xr2 = xr_call(h1, W2r, b2)

    out = pl.pallas_call(
        _tail_body,
        grid=(_GRID,),
        in_specs=[blk_a, blk_c, blk_r, blk_w,
                  pl.BlockSpec((1, 1, _BN), lambda i: (i, 0, 0)),
                  pl.BlockSpec((B, H), lambda i: (0, 0)),
                  pl.BlockSpec((OUT, H), lambda i: (0, 0)),
                  pl.BlockSpec((OUT, H), lambda i: (0, 0)),
                  pl.BlockSpec((1, OUT), lambda i: (0, 0))],
        out_specs=pl.BlockSpec((B, OUT), lambda i: (0, 0)),
        out_shape=jax.ShapeDtypeStruct((B, OUT), jnp.float32),
        scratch_shapes=[pltpu.VMEM((B, H), jnp.float32),
                        pltpu.VMEM((B, 1), jnp.float32)],
    )(a2, cntw, xr2, W2l, batch3, embed, wa, wb, bl)
    return out


# speedup vs baseline: 1.0923x; 1.0923x over previous
import jax
import jax.numpy as jnp
from jax import lax
from jax.experimental import pallas as pl
from jax.experimental.pallas import tpu as pltpu
from jax.experimental.pallas import tpu_sc as plsc

N = 10000
E = 320000
D = 128
H = 128
B = 64
OUT = 64

NC = 2
NS = 16
NW = NC * NS
K = 64
K2 = 128
NCHUNK = 160
EPW = NCHUNK * K
EPAD = NW * EPW
LAST_TRIPS = (E - (NW - 1) * EPW) // K
NACC2 = N + 16
ROWS_PT = 624
TAIL_OFF = ROWS_PT * NS
TAIL = N - TAIL_OFF

_HIGH = lax.Precision.HIGHEST


def _make_seg_body(ph, k, with_cnt):
    nchunk = EPW // k
    last_trips = (E - (NW - 1) * EPW) // k
    def body(*refs):
        if with_cnt:
            (table, src2, dst2, z128, z1, acc_out, cnt_out,
             srcv, dstv, r0, r1, cntp, sem0, sem1, acc) = refs
        else:
            (table, src2, dst2, z128, acc_out,
             srcv, dstv, r0, r1, sem0, sem1, acc) = refs
        c = lax.axis_index("c")
        s = lax.axis_index("s")
        wid = s * NC + c
        roff = s * ROWS_PT
        pltpu.sync_copy(z128.at[pl.ds(roff, ROWS_PT)],
                        acc.at[pl.ds(roff, ROWS_PT)])
        if with_cnt:
            pltpu.sync_copy(z1, cntp)
            ones = jnp.ones((16,), jnp.float32)

        @pl.when(s == NS - 1)
        def _():
            pltpu.sync_copy(z128.at[pl.ds(TAIL_OFF, TAIL)],
                            acc.at[pl.ds(TAIL_OFF, TAIL)])

        plsc.subcore_barrier()

        trips = jnp.where(wid == NW - 1, last_trips, nchunk)

        for p in range(nchunk // ph):
            pltpu.sync_copy(src2.at[wid, pl.ds(p * ph, ph)], srcv)
            pltpu.sync_copy(dst2.at[wid, pl.ds(p * ph, ph)], dstv)
            t_p = jnp.clip(trips - p * ph, 0, ph)

            @pl.when(t_p > 0)
            def _():
                pltpu.async_copy(table.at[srcv.at[0]], r0, sem0)

            def pair(i, carry):
                j0 = 2 * i
                cp1 = pltpu.async_copy(table.at[srcv.at[j0 + 1]], r1, sem1)
                pltpu.make_async_copy(table.at[srcv.at[j0]], r0, sem0).wait()
                pltpu.sync_copy(r0, acc.at[dstv.at[j0]], add=True)

                @pl.when(j0 + 2 < t_p)
                def _():
                    pltpu.async_copy(table.at[srcv.at[j0 + 2]], r0, sem0)

                if with_cnt:
                    for q in range(k // 16):
                        idx = dstv[j0, pl.ds(q * 16, 16)]
                        plsc.addupdate_scatter(cntp, [idx], ones)
                cp1.wait()
                pltpu.sync_copy(r1, acc.at[dstv.at[j0 + 1]], add=True)
                if with_cnt:
                    for q in range(k // 16):
                        idx = dstv[j0 + 1, pl.ds(q * 16, 16)]
                        plsc.addupdate_scatter(cntp, [idx], ones)
                return carry

            lax.fori_loop(0, t_p // 2, pair, 0)
        plsc.subcore_barrier()
        pltpu.sync_copy(acc.at[pl.ds(roff, ROWS_PT)],
                        acc_out.at[c, pl.ds(roff, ROWS_PT)])
        if with_cnt:
            pltpu.sync_copy(cntp, cnt_out.at[wid, 0])

        @pl.when(s == NS - 1)
        def _():
            pltpu.sync_copy(acc.at[pl.ds(TAIL_OFF, TAIL)],
                            acc_out.at[c, pl.ds(TAIL_OFF, TAIL)])

    return body


def _make_sc_kernels():
    mesh = plsc.VectorSubcoreMesh(
        core_axis_name="c", subcore_axis_name="s",
        num_cores=NC, num_subcores=NS,
    )
    ph1 = 40
    seg1 = pl.kernel(
        _make_seg_body(ph1, K, True),
        out_type=(jax.ShapeDtypeStruct((NC, N, D), jnp.float32),
                  jax.ShapeDtypeStruct((NW, 1, NACC2), jnp.float32)),
        mesh=mesh,
        compiler_params=pltpu.CompilerParams(needs_layout_passes=False),
        scratch_types=[
            pltpu.VMEM((ph1, K), jnp.int32),
            pltpu.VMEM((ph1, K), jnp.int32),
            pltpu.VMEM((K, D), jnp.float32),
            pltpu.VMEM((K, D), jnp.float32),
            pltpu.VMEM((NACC2,), jnp.float32),
            pltpu.SemaphoreType.DMA,
            pltpu.SemaphoreType.DMA,
            pltpu.VMEM_SHARED((N, D), jnp.float32),
        ],
        name="seg_cnt",
    )
    ph2 = 16
    seg2 = pl.kernel(
        _make_seg_body(ph2, K2, False),
        out_type=jax.ShapeDtypeStruct((NC, N, D), jnp.float32),
        mesh=mesh,
        scratch_types=[
            pltpu.VMEM((ph2, K2), jnp.int32),
            pltpu.VMEM((ph2, K2), jnp.int32),
            pltpu.VMEM((K2, D), jnp.float32),
            pltpu.VMEM((K2, D), jnp.float32),
            pltpu.SemaphoreType.DMA,
            pltpu.SemaphoreType.DMA,
            pltpu.VMEM_SHARED((N, D), jnp.float32),
        ],
        name="seg_sum",
    )
    return seg1, seg2


_sc_cache = []


def _sc_kernels():
    if not _sc_cache:
        _sc_cache.append(_make_sc_kernels())
    return _sc_cache[0]


_GRID = 10
_BN = N // _GRID


def _xr_body(x_ref, w_ref, b_ref, xr_ref):
    xr_ref[...] = lax.dot_general(x_ref[...], w_ref[...], (((1,), (1,)), ((), ())),
                                  precision=_HIGH) + b_ref[...]


def _mid_body(a_ref, c_ref, xr1_ref, w1l_ref, h1_ref):
    cnt = jnp.sum(c_ref[...], axis=1)
    inv = 1.0 / jnp.maximum(cnt, 1.0)
    mean = (a_ref[0] + a_ref[1]) * inv[:, None]
    h1 = lax.dot_general(mean, w1l_ref[...], (((1,), (1,)), ((), ())),
                         precision=_HIGH) + xr1_ref[...]
    h1_ref[...] = jnp.maximum(h1, 0.0)


def _tail_body(a_ref, c_ref, xr2_ref, w2l_ref, batch_ref, embed_ref,
               wa_ref, wb_ref, blin_ref, out_ref, pool_ref, cb_ref):
    i = pl.program_id(0)
    cnt = jnp.sum(c_ref[...], axis=1)
    inv = 1.0 / jnp.maximum(cnt, 1.0)
    mean = (a_ref[0] + a_ref[1]) * inv[:, None]
    h2 = lax.dot_general(mean, w2l_ref[...], (((1,), (1,)), ((), ())),
                         precision=_HIGH) + xr2_ref[...]
    h2 = jnp.maximum(h2, 0.0)
    ids = lax.broadcasted_iota(jnp.int32, (B, _BN), 0)
    oneh = (ids == batch_ref[0]).astype(jnp.float32)
    pool_p = lax.dot_general(oneh, h2, (((1,), (0,)), ((), ())),
                             precision=_HIGH)
    cb_p = jnp.sum(oneh, axis=1, keepdims=True)

    @pl.when(i == 0)
    def _():
        pool_ref[...] = jnp.zeros_like(pool_ref)
        cb_ref[...] = jnp.zeros_like(cb_ref)

    pool_ref[...] += pool_p
    cb_ref[...] += cb_p

    @pl.when(i == _GRID - 1)
    def _():
        pooled = pool_ref[...] / jnp.maximum(cb_ref[...], 1.0)
        out = lax.dot_general(pooled, wa_ref[...], (((1,), (1,)), ((), ())),
                              precision=_HIGH)
        out = out + lax.dot_general(embed_ref[...], wb_ref[...],
                                    (((1,), (1,)), ((), ())), precision=_HIGH)
        out_ref[...] = out + blin_ref[...]


def kernel(x, edge_index, batch, embed, W1l, b1l, W1r, W2l, b2l, W2r, Wlin, blin):
    npad = EPAD - E
    srcf = jnp.concatenate([edge_index[0], jnp.zeros((npad,), jnp.int32)])
    dstf = jnp.concatenate([edge_index[1], jnp.zeros((npad,), jnp.int32)])
    src2 = srcf.reshape(NW, NCHUNK, K)
    dst2 = dstf.reshape(NW, NCHUNK, K)
    src2b = srcf.reshape(NW, EPW // K2, K2)
    dst2b = dstf.reshape(NW, EPW // K2, K2)
    z128 = jnp.zeros((N, D), jnp.float32)
    z1 = jnp.zeros((NACC2,), jnp.float32)
    b1 = b1l.reshape(1, H)
    b2 = b2l.reshape(1, H)
    bl = blin.reshape(1, OUT)
    batch3 = batch.reshape(_GRID, 1, _BN)
    wa = Wlin[:, :H]
    wb = Wlin[:, H:]

    seg1, seg2 = _sc_kernels()

    blk_a = pl.BlockSpec((NC, _BN, D), lambda i: (0, i, 0))
    blk_c = pl.BlockSpec((_BN, NW), lambda i: (i, 0))
    blk_r = pl.BlockSpec((_BN, D), lambda i: (i, 0))
    blk_w = pl.BlockSpec((H, H), lambda i: (0, 0))
    blk_b = pl.BlockSpec((1, H), lambda i: (0, 0))

    def xr_call(inp, w, bias):
        return pl.pallas_call(
            _xr_body,
            grid=(_GRID,),
            in_specs=[blk_r, blk_w, blk_b],
            out_specs=blk_r,
            out_shape=jax.ShapeDtypeStruct((N, D), jnp.float32),
        )(inp, w, bias)

    a1, cntw3 = seg1(x, src2, dst2, z128, z1)
    xr1 = xr_call(x, W1r, b1)
    cntw = jnp.transpose(cntw3.reshape(NW, NACC2))

    h1 = pl.pallas_call(
        _mid_body,
        grid=(_GRID,),
        in_specs=[blk_a, blk_c, blk_r, blk_w],
        out_specs=blk_r,
        out_shape=jax.ShapeDtypeStruct((N, D), jnp.float32),
    )(a1, cntw, xr1, W1l)

    a2 = seg2(h1, src2b, dst2b, z128)
    xr2 = xr_call(h1, W2r, b2)

    out = pl.pallas_call(
        _tail_body,
        grid=(_GRID,),
        in_specs=[blk_a, blk_c, blk_r, blk_w,
                  pl.BlockSpec((1, 1, _BN), lambda i: (i, 0, 0)),
                  pl.BlockSpec((B, H), lambda i: (0, 0)),
                  pl.BlockSpec((OUT, H), lambda i: (0, 0)),
                  pl.BlockSpec((OUT, H), lambda i: (0, 0)),
                  pl.BlockSpec((1, OUT), lambda i: (0, 0))],
        out_specs=pl.BlockSpec((B, OUT), lambda i: (0, 0)),
        out_shape=jax.ShapeDtypeStruct((B, OUT), jnp.float32),
        scratch_shapes=[pltpu.VMEM((B, H), jnp.float32),
                        pltpu.VMEM((B, 1), jnp.float32)],
    )(a2, cntw, xr2, W2l, batch3, embed, wa, wb, bl)
    return out


# speedup vs baseline: 1.1336x; 1.0378x over previous
import jax
import jax.numpy as jnp
from jax import lax
from jax.experimental import pallas as pl
from jax.experimental.pallas import tpu as pltpu
from jax.experimental.pallas import tpu_sc as plsc

N = 10000
E = 320000
D = 128
H = 128
B = 64
OUT = 64

NC = 2
NS = 16
NW = NC * NS
K = 64
K2 = 128
NCHUNK = 160
EPW = NCHUNK * K
EPAD = NW * EPW
LAST_TRIPS = (E - (NW - 1) * EPW) // K
NACC2 = N + 16
ROWS_PT = 624
TAIL_OFF = ROWS_PT * NS
TAIL = N - TAIL_OFF

_HIGH = lax.Precision.HIGHEST


def _make_seg_body(ph, k, with_cnt):
    nchunk = EPW // k
    last_trips = (E - (NW - 1) * EPW) // k
    def body(*refs):
        if with_cnt:
            (table, src2, dst2, z128, z1, acc_out, cnt_out,
             srcv, dstv, r0, r1, cntp, sem0, sem1, acc) = refs
        else:
            (table, src2, dst2, z128, acc_out,
             srcv, dstv, r0, r1, sem0, sem1, acc) = refs
        c = lax.axis_index("c")
        s = lax.axis_index("s")
        wid = s * NC + c
        roff = s * ROWS_PT
        pltpu.sync_copy(z128.at[pl.ds(roff, ROWS_PT)],
                        acc.at[pl.ds(roff, ROWS_PT)])
        if with_cnt:
            pltpu.sync_copy(z1, cntp)
            ones = jnp.ones((16,), jnp.float32)

        @pl.when(s == NS - 1)
        def _():
            pltpu.sync_copy(z128.at[pl.ds(TAIL_OFF, TAIL)],
                            acc.at[pl.ds(TAIL_OFF, TAIL)])

        plsc.subcore_barrier()

        trips = jnp.where(wid == NW - 1, last_trips, nchunk)

        for p in range(nchunk // ph):
            pltpu.sync_copy(src2.at[wid, pl.ds(p * ph, ph)], srcv)
            pltpu.sync_copy(dst2.at[wid, pl.ds(p * ph, ph)], dstv)
            t_p = jnp.clip(trips - p * ph, 0, ph)

            @pl.when(t_p > 0)
            def _():
                pltpu.async_copy(table.at[srcv.at[0]], r0, sem0)

            def pair(i, carry):
                j0 = 2 * i
                cp1 = pltpu.async_copy(table.at[srcv.at[j0 + 1]], r1, sem1)
                pltpu.make_async_copy(table.at[srcv.at[j0]], r0, sem0).wait()
                pltpu.sync_copy(r0, acc.at[dstv.at[j0]], add=True)

                @pl.when(j0 + 2 < t_p)
                def _():
                    pltpu.async_copy(table.at[srcv.at[j0 + 2]], r0, sem0)

                if with_cnt:
                    for q in range(k // 16):
                        idx = dstv[j0, pl.ds(q * 16, 16)]
                        plsc.addupdate_scatter(cntp, [idx], ones)
                cp1.wait()
                pltpu.sync_copy(r1, acc.at[dstv.at[j0 + 1]], add=True)
                if with_cnt:
                    for q in range(k // 16):
                        idx = dstv[j0 + 1, pl.ds(q * 16, 16)]
                        plsc.addupdate_scatter(cntp, [idx], ones)
                return carry

            lax.fori_loop(0, t_p // 2, pair, 0)
        plsc.subcore_barrier()
        pltpu.sync_copy(acc.at[pl.ds(roff, ROWS_PT)],
                        acc_out.at[c, pl.ds(roff, ROWS_PT)])
        if with_cnt:
            pltpu.sync_copy(cntp, cnt_out.at[wid, 0])

        @pl.when(s == NS - 1)
        def _():
            pltpu.sync_copy(acc.at[pl.ds(TAIL_OFF, TAIL)],
                            acc_out.at[c, pl.ds(TAIL_OFF, TAIL)])

    return body


def _make_sc_kernels():
    mesh = plsc.VectorSubcoreMesh(
        core_axis_name="c", subcore_axis_name="s",
        num_cores=NC, num_subcores=NS,
    )
    ph1 = 80
    seg1 = pl.kernel(
        _make_seg_body(ph1, K, True),
        out_type=(jax.ShapeDtypeStruct((NC, N, D), jnp.float32),
                  jax.ShapeDtypeStruct((NW, 1, NACC2), jnp.float32)),
        mesh=mesh,
        compiler_params=pltpu.CompilerParams(needs_layout_passes=False),
        scratch_types=[
            pltpu.VMEM((ph1, K), jnp.int32),
            pltpu.VMEM((ph1, K), jnp.int32),
            pltpu.VMEM((K, D), jnp.float32),
            pltpu.VMEM((K, D), jnp.float32),
            pltpu.VMEM((NACC2,), jnp.float32),
            pltpu.SemaphoreType.DMA,
            pltpu.SemaphoreType.DMA,
            pltpu.VMEM_SHARED((N, D), jnp.float32),
        ],
        name="seg_cnt",
    )
    ph2 = 40
    seg2 = pl.kernel(
        _make_seg_body(ph2, K2, False),
        out_type=jax.ShapeDtypeStruct((NC, N, D), jnp.float32),
        mesh=mesh,
        scratch_types=[
            pltpu.VMEM((ph2, K2), jnp.int32),
            pltpu.VMEM((ph2, K2), jnp.int32),
            pltpu.VMEM((K2, D), jnp.float32),
            pltpu.VMEM((K2, D), jnp.float32),
            pltpu.SemaphoreType.DMA,
            pltpu.SemaphoreType.DMA,
            pltpu.VMEM_SHARED((N, D), jnp.float32),
        ],
        name="seg_sum",
    )
    return seg1, seg2


_sc_cache = []


def _sc_kernels():
    if not _sc_cache:
        _sc_cache.append(_make_sc_kernels())
    return _sc_cache[0]


_GRID = 10
_BN = N // _GRID


def _xr_body(x_ref, w_ref, b_ref, xr_ref):
    xr_ref[...] = lax.dot_general(x_ref[...], w_ref[...], (((1,), (1,)), ((), ())),
                                  precision=_HIGH) + b_ref[...]


def _mid_body(a_ref, c_ref, xr1_ref, w1l_ref, h1_ref):
    cnt = jnp.sum(c_ref[...], axis=1)
    inv = 1.0 / jnp.maximum(cnt, 1.0)
    mean = (a_ref[0] + a_ref[1]) * inv[:, None]
    h1 = lax.dot_general(mean, w1l_ref[...], (((1,), (1,)), ((), ())),
                         precision=_HIGH) + xr1_ref[...]
    h1_ref[...] = jnp.maximum(h1, 0.0)


def _tail_body(a_ref, c_ref, xr2_ref, w2l_ref, batch_ref, embed_ref,
               wa_ref, wb_ref, blin_ref, out_ref, pool_ref, cb_ref):
    i = pl.program_id(0)
    cnt = jnp.sum(c_ref[...], axis=1)
    inv = 1.0 / jnp.maximum(cnt, 1.0)
    mean = (a_ref[0] + a_ref[1]) * inv[:, None]
    h2 = lax.dot_general(mean, w2l_ref[...], (((1,), (1,)), ((), ())),
                         precision=_HIGH) + xr2_ref[...]
    h2 = jnp.maximum(h2, 0.0)
    ids = lax.broadcasted_iota(jnp.int32, (B, _BN), 0)
    oneh = (ids == batch_ref[0]).astype(jnp.float32)
    pool_p = lax.dot_general(oneh, h2, (((1,), (0,)), ((), ())),
                             precision=_HIGH)
    cb_p = jnp.sum(oneh, axis=1, keepdims=True)

    @pl.when(i == 0)
    def _():
        pool_ref[...] = jnp.zeros_like(pool_ref)
        cb_ref[...] = jnp.zeros_like(cb_ref)

    pool_ref[...] += pool_p
    cb_ref[...] += cb_p

    @pl.when(i == _GRID - 1)
    def _():
        pooled = pool_ref[...] / jnp.maximum(cb_ref[...], 1.0)
        out = lax.dot_general(pooled, wa_ref[...], (((1,), (1,)), ((), ())),
                              precision=_HIGH)
        out = out + lax.dot_general(embed_ref[...], wb_ref[...],
                                    (((1,), (1,)), ((), ())), precision=_HIGH)
        out_ref[...] = out + blin_ref[...]


def kernel(x, edge_index, batch, embed, W1l, b1l, W1r, W2l, b2l, W2r, Wlin, blin):
    npad = EPAD - E
    srcf = jnp.concatenate([edge_index[0], jnp.zeros((npad,), jnp.int32)])
    dstf = jnp.concatenate([edge_index[1], jnp.zeros((npad,), jnp.int32)])
    src2 = srcf.reshape(NW, NCHUNK, K)
    dst2 = dstf.reshape(NW, NCHUNK, K)
    src2b = srcf.reshape(NW, EPW // K2, K2)
    dst2b = dstf.reshape(NW, EPW // K2, K2)
    z128 = jnp.zeros((N, D), jnp.float32)
    z1 = jnp.zeros((NACC2,), jnp.float32)
    b1 = b1l.reshape(1, H)
    b2 = b2l.reshape(1, H)
    bl = blin.reshape(1, OUT)
    batch3 = batch.reshape(_GRID, 1, _BN)
    wa = Wlin[:, :H]
    wb = Wlin[:, H:]

    seg1, seg2 = _sc_kernels()

    blk_a = pl.BlockSpec((NC, _BN, D), lambda i: (0, i, 0))
    blk_c = pl.BlockSpec((_BN, NW), lambda i: (i, 0))
    blk_r = pl.BlockSpec((_BN, D), lambda i: (i, 0))
    blk_w = pl.BlockSpec((H, H), lambda i: (0, 0))
    blk_b = pl.BlockSpec((1, H), lambda i: (0, 0))

    def xr_call(inp, w, bias):
        return pl.pallas_call(
            _xr_body,
            grid=(_GRID,),
            in_specs=[blk_r, blk_w, blk_b],
            out_specs=blk_r,
            out_shape=jax.ShapeDtypeStruct((N, D), jnp.float32),
        )(inp, w, bias)

    a1, cntw3 = seg1(x, src2, dst2, z128, z1)
    xr1 = xr_call(x, W1r, b1)
    cntw = jnp.transpose(cntw3.reshape(NW, NACC2))

    h1 = pl.pallas_call(
        _mid_body,
        grid=(_GRID,),
        in_specs=[blk_a, blk_c, blk_r, blk_w],
        out_specs=blk_r,
        out_shape=jax.ShapeDtypeStruct((N, D), jnp.float32),
    )(a1, cntw, xr1, W1l)

    a2 = seg2(h1, src2b, dst2b, z128)
    xr2 = xr_call(h1, W2r, b2)

    out = pl.pallas_call(
        _tail_body,
        grid=(_GRID,),
        in_specs=[blk_a, blk_c, blk_r, blk_w,
                  pl.BlockSpec((1, 1, _BN), lambda i: (i, 0, 0)),
                  pl.BlockSpec((B, H), lambda i: (0, 0)),
                  pl.BlockSpec((OUT, H), lambda i: (0, 0)),
                  pl.BlockSpec((OUT, H), lambda i: (0, 0)),
                  pl.BlockSpec((1, OUT), lambda i: (0, 0))],
        out_specs=pl.BlockSpec((B, OUT), lambda i: (0, 0)),
        out_shape=jax.ShapeDtypeStruct((B, OUT), jnp.float32),
        scratch_shapes=[pltpu.VMEM((B, H), jnp.float32),
                        pltpu.VMEM((B, 1), jnp.float32)],
    )(a2, cntw, xr2, W2l, batch3, embed, wa, wb, bl)
    return out
